# Initial kernel scaffold; baseline (speedup 1.0000x reference)
import jax
import jax.numpy as jnp
from jax.experimental import pallas as pl


def kernel(loc_data, conf_data, prior_data):
    raise NotImplementedError("write your pallas kernel here")


# speedup vs baseline: 17.1317x; 17.1317x over previous
import jax
import jax.numpy as jnp
from jax.experimental import pallas as pl

R, C = 8, 2500
CR, CC = 8, 32
K = 200
CONF_THRESH_ = 0.01
NMS_THRESH_ = 0.45
NEG_INF = float("-inf")


def _ssd_body(loc_ref, pri_ref, cs_ref,
              os_ref, ox1_ref, oy1_ref, ox2_ref, oy2_ref):
    lx = loc_ref[0]
    ly = loc_ref[1]
    lw = loc_ref[2]
    lh = loc_ref[3]
    px = pri_ref[0]
    py = pri_ref[1]
    pw = pri_ref[2]
    ph = pri_ref[3]
    s = cs_ref[...]

    w = pw * jnp.exp(lw * 0.2)
    h = ph * jnp.exp(lh * 0.2)
    x1 = (px + (lx * 0.1) * pw) - w / 2.0
    y1 = (py + (ly * 0.1) * ph) - h / 2.0
    x2 = w + x1
    y2 = h + y1

    masked = jnp.where(s > CONF_THRESH_, s, NEG_INF)
    rows_i = jax.lax.broadcasted_iota(jnp.int32, (R, C), 0)
    cols_i = jax.lax.broadcasted_iota(jnp.int32, (R, C), 1)
    idx2d = rows_i * C + cols_i

    trows = jax.lax.broadcasted_iota(jnp.int32, (CR, CC), 0)
    tcols = jax.lax.broadcasted_iota(jnp.int32, (CR, CC), 1)
    tpos = trows * CC + tcols

    zc = jnp.zeros((CR, CC), jnp.float32)

    def sel_body(t, carry):
        ms, a_s, a_x1, a_y1, a_x2, a_y2, a_val = carry
        m = jnp.max(ms)
        eqm = ms == m
        pos = jnp.max(jnp.where(eqm, idx2d, -1))
        sel = idx2d == pos
        vs = jnp.sum(jnp.where(sel, s, 0.0))
        vx1 = jnp.sum(jnp.where(sel, x1, 0.0))
        vy1 = jnp.sum(jnp.where(sel, y1, 0.0))
        vx2 = jnp.sum(jnp.where(sel, x2, 0.0))
        vy2 = jnp.sum(jnp.where(sel, y2, 0.0))
        selt = tpos == t
        a_s = jnp.where(selt, vs, a_s)
        a_x1 = jnp.where(selt, vx1, a_x1)
        a_y1 = jnp.where(selt, vy1, a_y1)
        a_x2 = jnp.where(selt, vx2, a_x2)
        a_y2 = jnp.where(selt, vy2, a_y2)
        a_val = jnp.where(selt & (m > CONF_THRESH_), 1, a_val)
        ms = jnp.where(sel, NEG_INF, ms)
        return ms, a_s, a_x1, a_y1, a_x2, a_y2, a_val

    carry0 = (masked, zc, zc, zc, zc, zc, jnp.zeros((CR, CC), jnp.int32))
    _, c_s, c_x1, c_y1, c_x2, c_y2, c_val = jax.lax.fori_loop(
        0, K, sel_body, carry0)

    area = (c_x2 - c_x1) * (c_y2 - c_y1)

    def nms_body(t, carry):
        active, r_s, r_x1, r_y1, r_x2, r_y2 = carry
        any_active = jnp.max(active) > 0
        p = jnp.min(jnp.where(active > 0, tpos, CR * CC))
        sel = tpos == p
        ps = jnp.sum(jnp.where(sel, c_s, 0.0))
        px1 = jnp.sum(jnp.where(sel, c_x1, 0.0))
        py1 = jnp.sum(jnp.where(sel, c_y1, 0.0))
        px2 = jnp.sum(jnp.where(sel, c_x2, 0.0))
        py2 = jnp.sum(jnp.where(sel, c_y2, 0.0))
        parea = jnp.sum(jnp.where(sel, area, 0.0))
        xx1 = jnp.maximum(c_x1, px1)
        yy1 = jnp.maximum(c_y1, py1)
        xx2 = jnp.minimum(c_x2, px2)
        yy2 = jnp.minimum(c_y2, py2)
        wv = jnp.maximum(xx2 - xx1, 0.0)
        hv = jnp.maximum(yy2 - yy1, 0.0)
        inter = wv * hv
        union = area - inter + parea
        iou = inter / union
        new_active = jnp.where((iou <= NMS_THRESH_) & jnp.logical_not(sel),
                               active, 0)
        selt = (tpos == t) & any_active
        r_s = jnp.where(selt, ps, r_s)
        r_x1 = jnp.where(selt, px1, r_x1)
        r_y1 = jnp.where(selt, py1, r_y1)
        r_x2 = jnp.where(selt, px2, r_x2)
        r_y2 = jnp.where(selt, py2, r_y2)
        active = jnp.where(any_active, new_active, active)
        return active, r_s, r_x1, r_y1, r_x2, r_y2

    _, r_s, r_x1, r_y1, r_x2, r_y2 = jax.lax.fori_loop(
        0, K, nms_body, (c_val, zc, zc, zc, zc, zc))

    os_ref[...] = r_s
    ox1_ref[...] = r_x1
    oy1_ref[...] = r_y1
    ox2_ref[...] = r_x2
    oy2_ref[...] = r_y2


def kernel(loc_data, conf_data, prior_data):
    loc_t = loc_data[0].T.reshape(4, R, C)
    pri_t = prior_data.T.reshape(4, R, C)
    cs = conf_data[0, :, 1].reshape(R, C)

    outs = pl.pallas_call(
        _ssd_body,
        out_shape=[jax.ShapeDtypeStruct((CR, CC), jnp.float32)] * 5,
    )(loc_t, pri_t, cs)

    r_s, r_x1, r_y1, r_x2, r_y2 = outs
    rows = jnp.stack(
        [r_s.reshape(-1), r_x1.reshape(-1), r_y1.reshape(-1),
         r_x2.reshape(-1), r_y2.reshape(-1)], axis=1)[:K]
    out = jnp.zeros((1, 2, K, 5), jnp.float32)
    return out.at[0, 1].set(rows)
